# baseline (speedup 1.0000x reference)
import jax
import jax.numpy as jnp
from jax.experimental import pallas as pl
from jax.experimental.pallas import tpu as pltpu

_TILE = 1024


def _fused_body(s_ref, lnw_ref, lnb_ref, w1_ref, b1_ref, w2_ref, b2_ref,
                w3_ref, b3_ref, xl_ref, xr_ref, o_ref):
    k = xl_ref.shape[1]
    gl = jnp.dot(xl_ref[...].astype(jnp.bfloat16), s_ref[:k],
                 preferred_element_type=jnp.float32)
    gr = jnp.dot(xr_ref[...].astype(jnp.bfloat16), s_ref[k:],
                 preferred_element_type=jnp.float32)
    g = gl + gr
    mu = jnp.mean(g, axis=-1, keepdims=True)
    var = jnp.mean(g * g, axis=-1, keepdims=True) - mu * mu
    g = (g - mu) * jax.lax.rsqrt(var + 1e-5) * lnw_ref[...] + lnb_ref[...]
    h = jnp.dot(g.astype(jnp.bfloat16), w1_ref[...],
                preferred_element_type=jnp.float32) + b1_ref[...]
    h = jnp.dot(h.astype(jnp.bfloat16), w2_ref[...],
                preferred_element_type=jnp.float32) + b2_ref[...]
    h = jax.nn.sigmoid(h)
    o_ref[...] = jnp.dot(h.astype(jnp.bfloat16), w3_ref[...],
                         preferred_element_type=jnp.float32) + b3_ref[...]


def kernel(x, sparse_values, ln_w, ln_b, W1, b1, W2, b2, W3, b3,
           row_idx, col_idx):
    B, input_dim = x.shape
    n_blocks = ln_w.shape[0]
    half = input_dim // 2
    onehot = (col_idx[:, None] == jnp.arange(n_blocks, dtype=col_idx.dtype)[None, :])
    S = jnp.where(onehot, sparse_values[:, None], jnp.float32(0)).astype(jnp.bfloat16)
    W1 = W1.astype(jnp.bfloat16)
    W2 = W2.astype(jnp.bfloat16)
    W3 = W3.astype(jnp.bfloat16)

    grid = (B // _TILE,)
    full = lambda shape: pl.BlockSpec(shape, lambda i: (0,) * len(shape))
    out = pl.pallas_call(
        _fused_body,
        grid=grid,
        in_specs=[
            full((input_dim, n_blocks)),
            full((n_blocks,)),
            full((n_blocks,)),
            full((n_blocks, W1.shape[1])),
            full((W1.shape[1],)),
            full((W2.shape[0], W2.shape[1])),
            full((W2.shape[1],)),
            full((W3.shape[0], W3.shape[1])),
            full((W3.shape[1],)),
            pl.BlockSpec((_TILE, half), lambda i: (i, 0)),
            pl.BlockSpec((_TILE, half), lambda i: (i, 1)),
        ],
        out_specs=pl.BlockSpec((_TILE, 1), lambda i: (i, 0)),
        out_shape=jax.ShapeDtypeStruct((B, 1), jnp.float32),
        compiler_params=pltpu.CompilerParams(
            dimension_semantics=("parallel",)),
    )(S, ln_w, ln_b, W1, b1, W2, b2, W3, b3, x, x)
    return out


# speedup vs baseline: 1.0252x; 1.0252x over previous
import jax
import jax.numpy as jnp
from jax.experimental import pallas as pl
from jax.experimental.pallas import tpu as pltpu

_TILE = 2048


def _fused_body(s_ref, lnw_ref, lnb_ref, w1_ref, b1_ref, w2_ref, b2_ref,
                w3_ref, b3_ref, xl_ref, xr_ref, o_ref):
    k = xl_ref.shape[1]
    gl = jnp.dot(xl_ref[...].astype(jnp.bfloat16), s_ref[:k],
                 preferred_element_type=jnp.float32)
    gr = jnp.dot(xr_ref[...].astype(jnp.bfloat16), s_ref[k:],
                 preferred_element_type=jnp.float32)
    g = gl + gr
    mu = jnp.mean(g, axis=-1, keepdims=True)
    var = jnp.mean(g * g, axis=-1, keepdims=True) - mu * mu
    g = (g - mu) * jax.lax.rsqrt(var + 1e-5) * lnw_ref[...] + lnb_ref[...]
    h = jnp.dot(g.astype(jnp.bfloat16), w1_ref[...],
                preferred_element_type=jnp.float32) + b1_ref[...]
    h = jnp.dot(h.astype(jnp.bfloat16), w2_ref[...],
                preferred_element_type=jnp.float32) + b2_ref[...]
    h = 0.5 * jnp.tanh(0.5 * h) + 0.5
    o_ref[...] = jnp.dot(h.astype(jnp.bfloat16), w3_ref[...],
                         preferred_element_type=jnp.float32) + b3_ref[...]


def kernel(x, sparse_values, ln_w, ln_b, W1, b1, W2, b2, W3, b3,
           row_idx, col_idx):
    B, input_dim = x.shape
    n_blocks = ln_w.shape[0]
    half = input_dim // 2
    onehot = (col_idx[:, None] == jnp.arange(n_blocks, dtype=col_idx.dtype)[None, :])
    S = jnp.where(onehot, sparse_values[:, None], jnp.float32(0)).astype(jnp.bfloat16)
    W1 = W1.astype(jnp.bfloat16)
    W2 = W2.astype(jnp.bfloat16)
    W3 = W3.astype(jnp.bfloat16)

    grid = (B // _TILE,)
    full = lambda shape: pl.BlockSpec(shape, lambda i: (0,) * len(shape))
    out = pl.pallas_call(
        _fused_body,
        grid=grid,
        in_specs=[
            full((input_dim, n_blocks)),
            full((n_blocks,)),
            full((n_blocks,)),
            full((n_blocks, W1.shape[1])),
            full((W1.shape[1],)),
            full((W2.shape[0], W2.shape[1])),
            full((W2.shape[1],)),
            full((W3.shape[0], W3.shape[1])),
            full((W3.shape[1],)),
            pl.BlockSpec((_TILE, half), lambda i: (i, 0)),
            pl.BlockSpec((_TILE, half), lambda i: (i, 1)),
        ],
        out_specs=pl.BlockSpec((_TILE, 1), lambda i: (i, 0)),
        out_shape=jax.ShapeDtypeStruct((B, 1), jnp.float32),
        compiler_params=pltpu.CompilerParams(
            dimension_semantics=("parallel",)),
    )(S, ln_w, ln_b, W1, b1, W2, b2, W3, b3, x, x)
    return out
